# baseline (speedup 1.0000x reference)
import functools

import jax
import jax.numpy as jnp
from jax import lax
from jax.experimental import pallas as pl
from jax.experimental.pallas import tpu as pltpu
from jax.experimental.pallas import tpu_sc as plsc

N = 100000
DEG = 16
E = N * DEG
D_IN = 128
D_OUT = 128
H = 16

NW = 32
NODES_PW = N // NW
EDGES_PW = NODES_PW * DEG
CHUNK_E = 128
NPG = CHUNK_E // DEG
NCHUNK = EDGES_PW // CHUNK_E
TAIL_E = EDGES_PW - NCHUNK * CHUNK_E
TAIL_N = TAIL_E // DEG
NBUF = 6


def _nn(w):
    return jnp.where(w > 0, w + 1.0, jnp.exp(w))


BN1 = 8192
BE1 = BN1 * DEG


def _projk_body(x_ref, adj_ref, wk_ref, k_ref, src_ref):
    xb = x_ref[...]
    wk = _nn(wk_ref[...]) * (1.0 / D_IN)
    dn = (((1,), (1,)), ((), ()))
    kb = lax.dot_general(xb, wk, dn, preferred_element_type=jnp.float32)
    k_ref[...] = jnp.concatenate(
        [kb, jnp.zeros((BN1, D_IN - H), jnp.float32)], axis=1
    )
    src_ref[...] = adj_ref[0] * 8


def _proj_k(x, adj, Wk):
    grid = ((N + BN1 - 1) // BN1,)
    return pl.pallas_call(
        _projk_body,
        grid=grid,
        in_specs=[
            pl.BlockSpec((BN1, D_IN), lambda i: (i, 0)),
            pl.BlockSpec((2, BE1), lambda i: (0, i)),
            pl.BlockSpec((H, D_IN), lambda i: (0, 0)),
        ],
        out_specs=[
            pl.BlockSpec((BN1, D_IN), lambda i: (i, 0)),
            pl.BlockSpec((BE1,), lambda i: (i,)),
        ],
        out_shape=[
            jax.ShapeDtypeStruct((N, D_IN), jnp.float32),
            jax.ShapeDtypeStruct((E,), jnp.int32),
        ],
    )(x, adj, Wk)


def _projq_body(x_ref, wq_ref, q_ref):
    wq = _nn(wq_ref[...]) * (1.0 / D_IN)
    dn = (((1,), (1,)), ((), ()))
    q_ref[...] = lax.dot_general(
        x_ref[...], wq, dn, preferred_element_type=jnp.float32
    )


def _proj_q(x, Wq):
    grid = ((N + BN1 - 1) // BN1,)
    return pl.pallas_call(
        _projq_body,
        grid=grid,
        in_specs=[
            pl.BlockSpec((BN1, D_IN), lambda i: (i, 0)),
            pl.BlockSpec((H, D_IN), lambda i: (0, 0)),
        ],
        out_specs=pl.BlockSpec((BN1, H), lambda i: (i, 0)),
        out_shape=jax.ShapeDtypeStruct((N, H), jnp.float32),
    )(x, Wq)


_sc_mesh = plsc.VectorSubcoreMesh(core_axis_name="c", subcore_axis_name="s")


@functools.partial(
    pl.kernel,
    mesh=_sc_mesh,
    compiler_params=pltpu.CompilerParams(use_tc_tiling_on_sc=False),
    out_type=jax.ShapeDtypeStruct((N, D_IN), jnp.float32),
    scratch_types=[
        pltpu.VMEM((EDGES_PW,), jnp.int32),
        pltpu.VMEM((NBUF, CHUNK_E, H), jnp.float32),
        pltpu.VMEM((TAIL_E, H), jnp.float32),
        pltpu.VMEM((NODES_PW, H), jnp.float32),
        pltpu.SemaphoreType.DMA,
        pltpu.SemaphoreType.DMA,
        pltpu.SemaphoreType.DMA,
        pltpu.SemaphoreType.DMA,
        pltpu.SemaphoreType.DMA,
        pltpu.SemaphoreType.DMA,
    ],
)
def _sc_segsum(ktab, src, seg, idx_v, rows_v, tail_v, acc_v, *sems):
    wid = lax.axis_index("s") * 2 + lax.axis_index("c")
    pltpu.sync_copy(src.at[pl.ds(wid * EDGES_PW, EDGES_PW)], idx_v)
    for b in range(NBUF):
        pltpu.async_copy(
            ktab.at[idx_v.at[pl.ds(b * CHUNK_E, CHUNK_E)]], rows_v.at[b], sems[b]
        )

    def step(g, _):
        for b in range(NBUF):
            t = g * NBUF + b
            pltpu.make_async_copy(
                ktab.at[pl.ds(0, CHUNK_E)], rows_v.at[b], sems[b]
            ).wait()
            for i in range(NPG):
                vs = [rows_v[b, DEG * i + j] for j in range(DEG)]
                while len(vs) > 1:
                    vs = [vs[p] + vs[p + 1] for p in range(0, len(vs), 2)]
                acc_v[t * NPG + i] = vs[0]
            nxt = t + NBUF

            @pl.when(nxt < NCHUNK)
            def _():
                pltpu.async_copy(
                    ktab.at[idx_v.at[pl.ds(nxt * CHUNK_E, CHUNK_E)]],
                    rows_v.at[b],
                    sems[b],
                )

        return 0

    lax.fori_loop(0, NCHUNK // NBUF, step, 0)
    pltpu.async_copy(
        ktab.at[idx_v.at[pl.ds(NCHUNK * CHUNK_E, TAIL_E)]], tail_v, sems[0]
    ).wait()
    for i in range(TAIL_N):
        vs = [tail_v[DEG * i + j] for j in range(DEG)]
        while len(vs) > 1:
            vs = [vs[p] + vs[p + 1] for p in range(0, len(vs), 2)]
        acc_v[NCHUNK * NPG + i] = vs[0]
    pltpu.sync_copy(
        acc_v,
        seg.at[pl.ds(wid * NODES_PW, NODES_PW), pl.ds(0, H)],
    )


BN3 = 8192


def _out_body(q_ref, seg_ref, we_ref, wv_ref, b_ref, o_ref):
    q = q_ref[...]
    seg = seg_ref[:, :H]
    ss = _nn(we_ref[...]) * q * q + q * seg * (1.0 / DEG)
    norm = jnp.sum(ss, axis=1, keepdims=True) + 1e-9
    attn = ss / norm
    wv = _nn(wv_ref[...])
    dn = (((1,), (1,)), ((), ()))
    o_ref[...] = lax.dot_general(
        attn, wv, dn, preferred_element_type=jnp.float32
    ) + _nn(b_ref[...])


def _final(q_emb, seg, w_ego, Wv, bias):
    grid = ((N + BN3 - 1) // BN3,)
    return pl.pallas_call(
        _out_body,
        grid=grid,
        in_specs=[
            pl.BlockSpec((BN3, H), lambda i: (i, 0)),
            pl.BlockSpec((BN3, D_IN), lambda i: (i, 0)),
            pl.BlockSpec((1, H), lambda i: (0, 0)),
            pl.BlockSpec((D_OUT, H), lambda i: (0, 0)),
            pl.BlockSpec((1, D_OUT), lambda i: (0, 0)),
        ],
        out_specs=pl.BlockSpec((BN3, D_OUT), lambda i: (i, 0)),
        out_shape=jax.ShapeDtypeStruct((N, D_OUT), jnp.float32),
    )(q_emb, seg, w_ego, Wv, bias)


def kernel(adj_list, x, Wq, Wk, w_ego, Wv, bias):
    k_big, src = _proj_k(x, adj_list, Wk)
    seg = _sc_segsum(k_big.reshape(N * NPG, H), src)
    q_emb = _proj_q(x, Wq)
    return _final(q_emb, seg, w_ego, Wv, bias)


# speedup vs baseline: 1.1235x; 1.1235x over previous
import functools

import jax
import jax.numpy as jnp
from jax import lax
from jax.experimental import pallas as pl
from jax.experimental.pallas import tpu as pltpu
from jax.experimental.pallas import tpu_sc as plsc

N = 100000
DEG = 16
E = N * DEG
D_IN = 128
D_OUT = 128
H = 16

NW = 32
NODES_PW = N // NW
EDGES_PW = NODES_PW * DEG
CHUNK_E = 128
NPG = CHUNK_E // DEG
NCHUNK = EDGES_PW // CHUNK_E
TAIL_E = EDGES_PW - NCHUNK * CHUNK_E
TAIL_N = TAIL_E // DEG
NBUF = 6


def _nn(w):
    return jnp.where(w > 0, w + 1.0, jnp.exp(w))


BN1 = 8192
BE1 = BN1 * DEG


def _projk_body(x_ref, adj_ref, wk_ref, k_ref, src_ref):
    xb = x_ref[...]
    wk = _nn(wk_ref[...]) * (1.0 / D_IN)
    dn = (((1,), (1,)), ((), ()))
    kb = lax.dot_general(xb, wk, dn, preferred_element_type=jnp.float32)
    k_ref[...] = jnp.concatenate(
        [kb, jnp.zeros((BN1, D_IN - H), jnp.float32)], axis=1
    )
    src_ref[...] = adj_ref[0] * 8


def _proj_k(x, adj, Wk):
    grid = ((N + BN1 - 1) // BN1,)
    return pl.pallas_call(
        _projk_body,
        grid=grid,
        in_specs=[
            pl.BlockSpec((BN1, D_IN), lambda i: (i, 0)),
            pl.BlockSpec((2, BE1), lambda i: (0, i)),
            pl.BlockSpec((H, D_IN), lambda i: (0, 0)),
        ],
        out_specs=[
            pl.BlockSpec((BN1, D_IN), lambda i: (i, 0)),
            pl.BlockSpec((BE1,), lambda i: (i,)),
        ],
        out_shape=[
            jax.ShapeDtypeStruct((N, D_IN), jnp.float32),
            jax.ShapeDtypeStruct((E,), jnp.int32),
        ],
    )(x, adj, Wk)


_sc_mesh = plsc.VectorSubcoreMesh(core_axis_name="c", subcore_axis_name="s")


@functools.partial(
    pl.kernel,
    mesh=_sc_mesh,
    compiler_params=pltpu.CompilerParams(use_tc_tiling_on_sc=False),
    out_type=jax.ShapeDtypeStruct((N, D_IN), jnp.float32),
    scratch_types=[
        pltpu.VMEM((EDGES_PW,), jnp.int32),
        pltpu.VMEM((NBUF, CHUNK_E, H), jnp.float32),
        pltpu.VMEM((TAIL_E, H), jnp.float32),
        pltpu.VMEM((NODES_PW, H), jnp.float32),
        pltpu.SemaphoreType.DMA,
        pltpu.SemaphoreType.DMA,
        pltpu.SemaphoreType.DMA,
        pltpu.SemaphoreType.DMA,
        pltpu.SemaphoreType.DMA,
        pltpu.SemaphoreType.DMA,
    ],
)
def _sc_segsum(ktab, src, seg, idx_v, rows_v, tail_v, acc_v, *sems):
    wid = lax.axis_index("s") * 2 + lax.axis_index("c")
    pltpu.sync_copy(src.at[pl.ds(wid * EDGES_PW, EDGES_PW)], idx_v)
    for b in range(NBUF):
        pltpu.async_copy(
            ktab.at[idx_v.at[pl.ds(b * CHUNK_E, CHUNK_E)]], rows_v.at[b], sems[b]
        )

    def step(g, _):
        for b in range(NBUF):
            t = g * NBUF + b
            pltpu.make_async_copy(
                ktab.at[pl.ds(0, CHUNK_E)], rows_v.at[b], sems[b]
            ).wait()
            for i in range(NPG):
                vs = [rows_v[b, DEG * i + j] for j in range(DEG)]
                while len(vs) > 1:
                    vs = [vs[p] + vs[p + 1] for p in range(0, len(vs), 2)]
                acc_v[t * NPG + i] = vs[0]
            nxt = t + NBUF

            @pl.when(nxt < NCHUNK)
            def _():
                pltpu.async_copy(
                    ktab.at[idx_v.at[pl.ds(nxt * CHUNK_E, CHUNK_E)]],
                    rows_v.at[b],
                    sems[b],
                )

        return 0

    lax.fori_loop(0, NCHUNK // NBUF, step, 0)
    pltpu.async_copy(
        ktab.at[idx_v.at[pl.ds(NCHUNK * CHUNK_E, TAIL_E)]], tail_v, sems[0]
    ).wait()
    for i in range(TAIL_N):
        vs = [tail_v[DEG * i + j] for j in range(DEG)]
        while len(vs) > 1:
            vs = [vs[p] + vs[p + 1] for p in range(0, len(vs), 2)]
        acc_v[NCHUNK * NPG + i] = vs[0]
    pltpu.sync_copy(
        acc_v,
        seg.at[pl.ds(wid * NODES_PW, NODES_PW), pl.ds(0, H)],
    )


BN3 = 8192


def _out_body(x_ref, seg_ref, wq_ref, we_ref, wv_ref, b_ref, o_ref):
    wq = _nn(wq_ref[...]) * (1.0 / D_IN)
    dn0 = (((1,), (1,)), ((), ()))
    q = lax.dot_general(
        x_ref[...], wq, dn0, preferred_element_type=jnp.float32
    )
    seg = seg_ref[:, :H]
    ss = _nn(we_ref[...]) * q * q + q * seg * (1.0 / DEG)
    norm = jnp.sum(ss, axis=1, keepdims=True) + 1e-9
    attn = ss / norm
    wv = _nn(wv_ref[...])
    dn = (((1,), (1,)), ((), ()))
    o_ref[...] = lax.dot_general(
        attn, wv, dn, preferred_element_type=jnp.float32
    ) + _nn(b_ref[...])


def _final(x, seg, Wq, w_ego, Wv, bias):
    grid = ((N + BN3 - 1) // BN3,)
    return pl.pallas_call(
        _out_body,
        grid=grid,
        in_specs=[
            pl.BlockSpec((BN3, D_IN), lambda i: (i, 0)),
            pl.BlockSpec((BN3, D_IN), lambda i: (i, 0)),
            pl.BlockSpec((H, D_IN), lambda i: (0, 0)),
            pl.BlockSpec((1, H), lambda i: (0, 0)),
            pl.BlockSpec((D_OUT, H), lambda i: (0, 0)),
            pl.BlockSpec((1, D_OUT), lambda i: (0, 0)),
        ],
        out_specs=pl.BlockSpec((BN3, D_OUT), lambda i: (i, 0)),
        out_shape=jax.ShapeDtypeStruct((N, D_OUT), jnp.float32),
    )(x, seg, Wq, w_ego, Wv, bias)


def kernel(adj_list, x, Wq, Wk, w_ego, Wv, bias):
    k_big, src = _proj_k(x, adj_list, Wk)
    seg = _sc_segsum(k_big.reshape(N * NPG, H), src)
    return _final(x, seg, Wq, w_ego, Wv, bias)
